# baseline (device time: 15421 ns/iter reference)
import jax
import jax.numpy as jnp
from jax import lax
from jax.experimental import pallas as pl
from jax.experimental.pallas import tpu as pltpu

N_DEV = 8
B, SQ, HQ, DH = 2, 128, 4, 64
BLK = 64
SCALE = 0.125


def kernel(x, Wq, K_ext, V_ext, Wo):
    d_model = x.shape[-1]

    K_t = jnp.transpose(K_ext, (0, 2, 3, 1)).reshape(B * HQ, DH, SQ)
    V_t = jnp.transpose(V_ext, (0, 2, 3, 1)).reshape(B * HQ, DH, SQ)

    SEND_ORDER = (1, 3, 4, 2, 5, 7, 6)

    def body(x_ref, wq_ref, k_ref, v_ref, wo_ref, out_ref, qbuf, sbuf,
             send_sems, recv_sems, credit_sems, k_scr, v_scr, copy_sems):
        my = lax.axis_index("i")
        barrier_sem = pltpu.get_barrier_semaphore()
        pl.semaphore_signal(barrier_sem, inc=1)
        pl.semaphore_wait(barrier_sem, 1)

        def rdma_pair(target):
            data = pltpu.make_async_remote_copy(
                src_ref=qbuf,
                dst_ref=qbuf,
                send_sem=send_sems.at[0, target],
                recv_sem=recv_sems.at[0],
                device_id=(target,),
                device_id_type=pl.DeviceIdType.MESH,
            )
            scales = pltpu.make_async_remote_copy(
                src_ref=sbuf,
                dst_ref=sbuf,
                send_sem=send_sems.at[1, target],
                recv_sem=recv_sems.at[1],
                device_id=(target,),
                device_id_type=pl.DeviceIdType.MESH,
            )
            return data, scales

        @pl.when(my == 0)
        def _():
            kcp = pltpu.make_async_copy(k_ref, k_scr, copy_sems.at[0])
            vcp = pltpu.make_async_copy(v_ref, v_scr, copy_sems.at[1])
            kcp.start()
            vcp.start()

            x2 = x_ref[...].astype(jnp.bfloat16).reshape(B * SQ, d_model)
            wq = wq_ref[...].astype(jnp.bfloat16)
            q2 = lax.dot_general(
                x2, wq, (((1,), (0,)), ((), ())),
                preferred_element_type=jnp.float32,
            ).astype(jnp.bfloat16)
            qT = jnp.transpose(
                q2.reshape(B, SQ, HQ, DH), (0, 2, 1, 3)
            ).reshape(B * HQ, SQ, DH)

            kcp.wait()
            vcp.wait()
            kT = k_scr[...].astype(jnp.bfloat16)
            vT = v_scr[...].astype(jnp.bfloat16)
            scores = lax.dot_general(
                qT, kT, (((2,), (1,)), ((0,), (0,))),
                preferred_element_type=jnp.float32,
            ) * SCALE
            rows = lax.broadcasted_iota(jnp.int32, (SQ, SQ), 0)
            cols = lax.broadcasted_iota(jnp.int32, (SQ, SQ), 1)
            keep = (cols // BLK) <= (rows // BLK)
            scores = jnp.where(keep[None], scores, -1e9)
            m = jnp.max(scores, axis=-1, keepdims=True)
            w = jnp.exp(scores - m)
            w = w / jnp.sum(w, axis=-1, keepdims=True)
            ctx = lax.dot_general(
                w.astype(jnp.bfloat16), vT, (((2,), (2,)), ((0,), (0,))),
                preferred_element_type=jnp.float32,
            ).astype(jnp.bfloat16)
            ctx2 = jnp.transpose(
                ctx.reshape(B, HQ, SQ, DH), (0, 2, 1, 3)
            ).reshape(B * SQ, HQ * DH)

            wo = wo_ref[...].astype(jnp.bfloat16)
            ob = lax.dot_general(
                ctx2, wo, (((1,), (0,)), ((), ())),
                preferred_element_type=jnp.float32,
            )
            for b in range(B):
                obb = ob[b * SQ:(b + 1) * SQ]
                out_ref[b] = obb.astype(jnp.bfloat16)
                rowmax = jnp.max(jnp.abs(obb), axis=-1, keepdims=True)
                qbuf[b] = jnp.round(obb * (127.0 / rowmax)).astype(jnp.int8)
                sbuf[pl.ds(b, 1), :] = jnp.transpose(rowmax)

            rdmas = []
            for t in SEND_ORDER:
                pl.semaphore_wait(credit_sems.at[t], 1)
                data, scales = rdma_pair(t)
                data.start()
                scales.start()
                rdmas += [data, scales]
            for r in rdmas:
                r.wait_send()

        @pl.when(my != 0)
        def _():
            pl.semaphore_signal(
                credit_sems.at[my], inc=1,
                device_id=(0,), device_id_type=pl.DeviceIdType.MESH,
            )
            data, scales = rdma_pair(0)
            data.wait_recv()
            scales.wait_recv()
            for b in range(B):
                s = sbuf[b, :] * (1.0 / 127.0)
                out_ref[b] = (
                    qbuf[b].astype(jnp.float32) * s[:, None]
                ).astype(jnp.bfloat16)

    return pl.pallas_call(
        body,
        out_shape=jax.ShapeDtypeStruct((B, SQ, d_model), jnp.bfloat16),
        in_specs=[
            pl.BlockSpec(memory_space=pltpu.VMEM),
            pl.BlockSpec(memory_space=pltpu.VMEM),
            pl.BlockSpec(memory_space=pltpu.MemorySpace.HBM),
            pl.BlockSpec(memory_space=pltpu.MemorySpace.HBM),
            pl.BlockSpec(memory_space=pltpu.VMEM),
        ],
        out_specs=pl.BlockSpec(memory_space=pltpu.VMEM),
        scratch_shapes=[
            pltpu.VMEM((B, SQ, d_model), jnp.int8),
            pltpu.VMEM((8, SQ), jnp.float32),
            pltpu.SemaphoreType.DMA((2, N_DEV)),
            pltpu.SemaphoreType.DMA((2,)),
            pltpu.SemaphoreType.REGULAR((N_DEV,)),
            pltpu.VMEM((B * HQ, DH, SQ), jnp.float32),
            pltpu.VMEM((B * HQ, DH, SQ), jnp.float32),
            pltpu.SemaphoreType.DMA((2,)),
        ],
        compiler_params=pltpu.CompilerParams(collective_id=0),
    )(x, Wq, K_t, V_t, Wo)


# device time: 10316 ns/iter; 1.4949x vs baseline; 1.4949x over previous
import jax
import jax.numpy as jnp
from jax import lax
from jax.experimental import pallas as pl
from jax.experimental.pallas import tpu as pltpu

N_DEV = 8
B, SQ, HQ, DH = 2, 128, 4, 64
BLK = 64
SCALE = 0.125


def kernel(x, Wq, K_ext, V_ext, Wo):
    d_model = x.shape[-1]

    ROOT_SENDS = (1, 4, 3, 7)
    RELAY = {3: 2, 4: 5, 7: 6}
    PARENT = {1: 0, 3: 0, 4: 0, 7: 0, 2: 3, 5: 4, 6: 7}

    def body(x_ref, wq_ref, k_ref, v_ref, wo_ref, out_ref, qbuf, sbuf,
             send_sems, recv_sems, credit_sems):
        my = lax.axis_index("i")
        barrier_sem = pltpu.get_barrier_semaphore()
        pl.semaphore_signal(barrier_sem, inc=1)
        pl.semaphore_wait(barrier_sem, 1)

        def rdma_pair(target):
            data = pltpu.make_async_remote_copy(
                src_ref=qbuf,
                dst_ref=qbuf,
                send_sem=send_sems.at[0, target],
                recv_sem=recv_sems.at[0],
                device_id=(target,),
                device_id_type=pl.DeviceIdType.MESH,
            )
            scales = pltpu.make_async_remote_copy(
                src_ref=sbuf,
                dst_ref=sbuf,
                send_sem=send_sems.at[1, target],
                recv_sem=recv_sems.at[1],
                device_id=(target,),
                device_id_type=pl.DeviceIdType.MESH,
            )
            return data, scales

        @pl.when(my == 0)
        def _():
            x2 = x_ref[...].astype(jnp.bfloat16).reshape(B * SQ, d_model)
            wq = wq_ref[...].astype(jnp.bfloat16)
            q2 = lax.dot_general(
                x2, wq, (((1,), (0,)), ((), ())),
                preferred_element_type=jnp.float32,
            ).astype(jnp.bfloat16)
            qT = jnp.transpose(
                q2.reshape(B, SQ, HQ, DH), (0, 2, 1, 3)
            ).reshape(B * HQ, SQ, DH)
            kT = jnp.transpose(
                k_ref[...].astype(jnp.bfloat16), (0, 2, 1, 3)
            ).reshape(B * HQ, SQ, DH)
            vT = jnp.transpose(
                v_ref[...].astype(jnp.bfloat16), (0, 2, 1, 3)
            ).reshape(B * HQ, SQ, DH)

            scores = lax.dot_general(
                qT, kT, (((2,), (2,)), ((0,), (0,))),
                preferred_element_type=jnp.float32,
            ) * SCALE
            rows = lax.broadcasted_iota(jnp.int32, (SQ, SQ), 0)
            cols = lax.broadcasted_iota(jnp.int32, (SQ, SQ), 1)
            keep = (cols // BLK) <= (rows // BLK)
            scores = jnp.where(keep[None], scores, -1e9)
            m = jnp.max(scores, axis=-1, keepdims=True)
            w = jnp.exp(scores - m)
            w = w / jnp.sum(w, axis=-1, keepdims=True)
            ctx = lax.dot_general(
                w.astype(jnp.bfloat16), vT, (((2,), (1,)), ((0,), (0,))),
                preferred_element_type=jnp.float32,
            ).astype(jnp.bfloat16)
            ctx2 = jnp.transpose(
                ctx.reshape(B, HQ, SQ, DH), (0, 2, 1, 3)
            ).reshape(B * SQ, HQ * DH)

            wo = wo_ref[...].astype(jnp.bfloat16)
            ob = lax.dot_general(
                ctx2, wo, (((1,), (0,)), ((), ())),
                preferred_element_type=jnp.float32,
            )
            for b in range(B):
                obb = ob[b * SQ:(b + 1) * SQ]
                out_ref[b] = obb.astype(jnp.bfloat16)
                rowmax = jnp.max(jnp.abs(obb), axis=-1, keepdims=True)
                qbuf[b] = jnp.round(obb * (127.0 / rowmax)).astype(jnp.int8)
                sbuf[pl.ds(b, 1), :] = jnp.transpose(rowmax)

            rdmas = []
            for t in ROOT_SENDS:
                pl.semaphore_wait(credit_sems.at[t], 1)
                data, scales = rdma_pair(t)
                data.start()
                scales.start()
                rdmas += [data, scales]
            for r in rdmas:
                r.wait_send()

        @pl.when(my != 0)
        def _():
            for dev, par in PARENT.items():
                @pl.when(my == dev)
                def _(par=par):
                    pl.semaphore_signal(
                        credit_sems.at[dev], inc=1,
                        device_id=(par,), device_id_type=pl.DeviceIdType.MESH,
                    )
            data, scales = rdma_pair(0)
            data.wait_recv()
            scales.wait_recv()

            for rly, leaf in RELAY.items():
                @pl.when(my == rly)
                def _(leaf=leaf):
                    pl.semaphore_wait(credit_sems.at[leaf], 1)
                    fdata, fscales = rdma_pair(leaf)
                    fdata.start()
                    fscales.start()

            for b in range(B):
                s = sbuf[b, :] * (1.0 / 127.0)
                out_ref[b] = (
                    qbuf[b].astype(jnp.float32) * s[:, None]
                ).astype(jnp.bfloat16)

            for rly, leaf in RELAY.items():
                @pl.when(my == rly)
                def _(leaf=leaf):
                    fdata, fscales = rdma_pair(leaf)
                    fdata.wait_send()
                    fscales.wait_send()

    return pl.pallas_call(
        body,
        out_shape=jax.ShapeDtypeStruct((B, SQ, d_model), jnp.bfloat16),
        in_specs=[pl.BlockSpec(memory_space=pltpu.VMEM)] * 5,
        out_specs=pl.BlockSpec(memory_space=pltpu.VMEM),
        scratch_shapes=[
            pltpu.VMEM((B, SQ, d_model), jnp.int8),
            pltpu.VMEM((8, SQ), jnp.float32),
            pltpu.SemaphoreType.DMA((2, N_DEV)),
            pltpu.SemaphoreType.DMA((2,)),
            pltpu.SemaphoreType.REGULAR((N_DEV,)),
        ],
        compiler_params=pltpu.CompilerParams(collective_id=0),
    )(x, Wq, K_ext, V_ext, Wo)
